# baseline (speedup 1.0000x reference)
import jax
import jax.numpy as jnp
from jax import lax
from jax.experimental import pallas as pl
from jax.experimental.pallas import tpu as pltpu


def _wta(po, n, C, thr):
    pot = po[:n]
    tn = po[n:]
    mx = jnp.max(pot, axis=1, keepdims=True)
    fired = mx > thr
    iota = lax.broadcasted_iota(jnp.int32, (n, C), 1)
    widx = jnp.min(jnp.where(pot == mx, iota, C), axis=1, keepdims=True)
    winner = iota == widx
    t = tn / jnp.maximum(pot, 1e-6)
    return jnp.where(winner & fired, t, 0.0)


def _pool(x, H, W, C):
    x = x.reshape(H // 2, 2, W, C).max(axis=1)
    return x.reshape(H // 2, W // 2, 2, C).max(axis=2)


def _net_body(c1_ref, w1_ref, w2_ref, w3_ref, o_ref, xp2_ref, xp3_ref,
              c2_ref, c3_ref):
    f32 = jnp.float32
    xp2_ref[:] = jnp.zeros(xp2_ref.shape, f32)
    xp3_ref[:] = jnp.zeros(xp3_ref.shape, f32)

    def l1_body(i, _):
        colsT = c1_ref[i]
        bothT = jnp.concatenate([(colsT > 0).astype(f32), colsT], axis=1)
        po = lax.dot_general(bothT, w1_ref[:], (((0,), (0,)), ((), ())),
                             preferred_element_type=f32)
        out = _wta(po, 2048, 30, 2.4)
        pooled = _pool(out, 16, 128, 30)
        xp2_ref[pl.ds(1 + i * 8, 8), 1:65, :] = pooled
        return 0

    lax.fori_loop(0, 8, l1_body, 0)

    def l2_body(i, _):
        for dy in range(3):
            for dx in range(3):
                piece = xp2_ref[pl.ds(i * 8 + dy, 8), dx:dx + 64, :]
                piece = piece.reshape(512, 30)
                j = (dy * 3 + dx) * 30
                c2_ref[:512, j:j + 30] = (piece > 0).astype(f32)
                c2_ref[512:, j:j + 30] = piece
        po = jnp.dot(c2_ref[:], w2_ref[:], preferred_element_type=f32)
        out = _wta(po, 512, 100, 1.0)
        pooled = _pool(out, 8, 64, 100)
        xp3_ref[pl.ds(1 + i * 4, 4), 1:33, :] = pooled
        return 0

    lax.fori_loop(0, 8, l2_body, 0)

    def l3_body(i, _):
        for dy in range(3):
            for dx in range(3):
                piece = xp3_ref[pl.ds(i * 8 + dy, 8), dx:dx + 32, :]
                piece = piece.reshape(256, 100)
                j = (dy * 3 + dx) * 100
                c3_ref[:256, j:j + 100] = (piece > 0).astype(f32)
                c3_ref[256:, j:j + 100] = piece
        po = jnp.dot(c3_ref[:], w3_ref[:], preferred_element_type=f32)
        out = _wta(po, 256, 200, 1.0)
        o_ref[pl.ds(i * 8, 8), :, :] = out.reshape(8, 32, 200)
        return 0

    lax.fori_loop(0, 4, l3_body, 0)


def kernel(spk_in, W1, W2, W3):
    xpad = jnp.pad(spk_in, ((0, 0), (2, 2), (2, 2)))
    cols1 = jnp.stack(
        [xpad[ic, dy:dy + 128, dx:dx + 128].reshape(128 * 128)
         for dy in range(5) for dx in range(5) for ic in range(2)])
    cols1 = cols1.reshape(50, 8, 2048).transpose(1, 0, 2)
    w1t = W1.transpose(2, 3, 1, 0).reshape(50, 30)
    w2t = W2.transpose(2, 3, 1, 0).reshape(270, 100)
    w3t = W3.transpose(2, 3, 1, 0).reshape(900, 200)
    out = pl.pallas_call(
        _net_body,
        out_shape=jax.ShapeDtypeStruct((32, 32, 200), jnp.float32),
        scratch_shapes=[
            pltpu.VMEM((66, 66, 30), jnp.float32),
            pltpu.VMEM((34, 34, 100), jnp.float32),
            pltpu.VMEM((2 * 8 * 64, 270), jnp.float32),
            pltpu.VMEM((2 * 8 * 32, 900), jnp.float32),
        ],
    )(cols1, w1t, w2t, w3t)
    return jnp.moveaxis(out, -1, 0)


# speedup vs baseline: 1.0630x; 1.0630x over previous
import jax
import jax.numpy as jnp
from jax import lax
from jax.experimental import pallas as pl
from jax.experimental.pallas import tpu as pltpu


def _wta(po, n, C, thr):
    pot = po[:n]
    tn = po[n:]
    mx = jnp.max(pot, axis=1, keepdims=True)
    fired = mx > thr
    iota = lax.broadcasted_iota(jnp.int32, (n, C), 1)
    widx = jnp.min(jnp.where(pot == mx, iota, C), axis=1, keepdims=True)
    winner = iota == widx
    t = tn / jnp.maximum(pot, 1e-6)
    return jnp.where(winner & fired, t, 0.0)


def _pool(x, H, W, C):
    x = x.reshape(H // 2, 2, W, C).max(axis=1)
    return x.reshape(H // 2, W // 2, 2, C).max(axis=2)


def _zero_border(ref, n, C):
    z = jnp.zeros((1, n + 2, C), jnp.float32)
    ref[0:1, :, :] = z
    ref[n + 1:n + 2, :, :] = z
    zc = jnp.zeros((n + 2, 1, C), jnp.float32)
    ref[:, 0:1, :] = zc
    ref[:, n + 1:n + 2, :] = zc


def _net_body(c1_ref, w1_ref, w2_ref, w3_ref, o_ref, xp2_ref, xp3_ref):
    f32 = jnp.float32
    _zero_border(xp2_ref, 64, 30)
    _zero_border(xp3_ref, 32, 100)

    def l1_body(i, _):
        colsT = c1_ref[i]
        bothT = jnp.concatenate([(colsT > 0).astype(f32), colsT], axis=1)
        po = lax.dot_general(bothT, w1_ref[:], (((0,), (0,)), ((), ())),
                             preferred_element_type=f32)
        out = _wta(po, 4096, 30, 2.4)
        pooled = _pool(out, 32, 128, 30)
        xp2_ref[pl.ds(1 + i * 16, 16), 1:65, :] = pooled
        return 0

    lax.fori_loop(0, 4, l1_body, 0)

    def l2_body(i, _):
        cols = jnp.concatenate(
            [xp2_ref[pl.ds(i * 8 + dy, 8), dx:dx + 64, :]
             for dy in range(3) for dx in range(3)], axis=-1)
        cols = cols.reshape(512, 270)
        both = jnp.concatenate([(cols > 0).astype(f32), cols], axis=0)
        po = jnp.dot(both, w2_ref[:], preferred_element_type=f32)
        out = _wta(po, 512, 100, 1.0)
        pooled = _pool(out, 8, 64, 100)
        xp3_ref[pl.ds(1 + i * 4, 4), 1:33, :] = pooled
        return 0

    lax.fori_loop(0, 8, l2_body, 0)

    def l3_body(i, _):
        cols = jnp.concatenate(
            [xp3_ref[pl.ds(i * 8 + dy, 8), dx:dx + 32, :]
             for dy in range(3) for dx in range(3)], axis=-1)
        cols = cols.reshape(256, 900)
        both = jnp.concatenate([(cols > 0).astype(f32), cols], axis=0)
        po = jnp.dot(both, w3_ref[:], preferred_element_type=f32)
        out = _wta(po, 256, 200, 1.0)
        o_ref[pl.ds(i * 8, 8), :, :] = out.reshape(8, 32, 200)
        return 0

    lax.fori_loop(0, 4, l3_body, 0)


def kernel(spk_in, W1, W2, W3):
    xpad = jnp.pad(spk_in, ((0, 0), (2, 2), (2, 2)))
    cols1 = jnp.stack(
        [xpad[ic, dy:dy + 128, dx:dx + 128].reshape(128 * 128)
         for dy in range(5) for dx in range(5) for ic in range(2)])
    cols1 = cols1.reshape(50, 4, 4096).transpose(1, 0, 2)
    w1t = W1.transpose(2, 3, 1, 0).reshape(50, 30)
    w2t = W2.transpose(2, 3, 1, 0).reshape(270, 100)
    w3t = W3.transpose(2, 3, 1, 0).reshape(900, 200)
    out = pl.pallas_call(
        _net_body,
        out_shape=jax.ShapeDtypeStruct((32, 32, 200), jnp.float32),
        scratch_shapes=[
            pltpu.VMEM((66, 66, 30), jnp.float32),
            pltpu.VMEM((34, 34, 100), jnp.float32),
        ],
    )(cols1, w1t, w2t, w3t)
    return jnp.moveaxis(out, -1, 0)


# speedup vs baseline: 1.0915x; 1.0268x over previous
import jax
import jax.numpy as jnp
from jax import lax
from jax.experimental import pallas as pl
from jax.experimental.pallas import tpu as pltpu


def _wta(po, n, C, thr):
    pot = po[:n]
    tn = po[n:]
    mx = jnp.max(pot, axis=1, keepdims=True)
    fired = mx > thr
    iota = lax.broadcasted_iota(jnp.int32, (n, C), 1)
    widx = jnp.min(jnp.where(pot == mx, iota, C), axis=1, keepdims=True)
    winner = iota == widx
    t = tn / jnp.maximum(pot, 1e-6)
    return jnp.where(winner & fired, t, 0.0)


def _pool(x, H, W, C):
    x = x.reshape(H // 2, 2, W, C).max(axis=1)
    return x.reshape(H // 2, W // 2, 2, C).max(axis=2)


def _zero_border(ref, n, C):
    z = jnp.zeros((1, n + 2, C), jnp.float32)
    ref[0:1, :, :] = z
    ref[n + 1:n + 2, :, :] = z
    zc = jnp.zeros((n + 2, 1, C), jnp.float32)
    ref[:, 0:1, :] = zc
    ref[:, n + 1:n + 2, :] = zc


def _net_body(c1_ref, w1_ref, w2_ref, w3_ref, o_ref, xp2_ref, xp3_ref):
    f32 = jnp.float32
    _zero_border(xp2_ref, 64, 30)
    _zero_border(xp3_ref, 32, 100)

    def l1_body(i, _):
        colsT = c1_ref[i]
        bothT = jnp.concatenate([(colsT > 0).astype(f32), colsT], axis=1)
        po = lax.dot_general(bothT, w1_ref[:], (((0,), (0,)), ((), ())),
                             preferred_element_type=f32)
        out = _wta(po, 4096, 30, 2.4)
        pooled = _pool(out, 32, 128, 30)
        xp2_ref[pl.ds(1 + i * 16, 16), 1:65, :] = pooled
        return 0

    lax.fori_loop(0, 4, l1_body, 0)

    def l2_body(i, _):
        cols = jnp.concatenate(
            [xp2_ref[pl.ds(i * 16 + dy, 16), dx:dx + 64, :]
             for dy in range(3) for dx in range(3)], axis=-1)
        cols = cols.reshape(1024, 270)
        both = jnp.concatenate([(cols > 0).astype(f32), cols], axis=0)
        po = jnp.dot(both, w2_ref[:], preferred_element_type=f32)
        out = _wta(po, 1024, 100, 1.0)
        pooled = _pool(out, 16, 64, 100)
        xp3_ref[pl.ds(1 + i * 8, 8), 1:33, :] = pooled
        return 0

    lax.fori_loop(0, 4, l2_body, 0)

    def l3_body(i, _):
        cols = jnp.concatenate(
            [xp3_ref[pl.ds(i * 16 + dy, 16), dx:dx + 32, :]
             for dy in range(3) for dx in range(3)], axis=-1)
        cols = cols.reshape(512, 900)
        both = jnp.concatenate([(cols > 0).astype(f32), cols], axis=0)
        po = jnp.dot(both, w3_ref[:], preferred_element_type=f32)
        out = _wta(po, 512, 200, 1.0)
        o_ref[pl.ds(i * 16, 16), :, :] = out.reshape(16, 32, 200)
        return 0

    lax.fori_loop(0, 2, l3_body, 0)


def kernel(spk_in, W1, W2, W3):
    xpad = jnp.pad(spk_in, ((0, 0), (2, 2), (2, 2)))
    cols1 = jnp.stack(
        [xpad[ic, dy:dy + 128, dx:dx + 128].reshape(128 * 128)
         for dy in range(5) for dx in range(5) for ic in range(2)])
    cols1 = cols1.reshape(50, 4, 4096).transpose(1, 0, 2)
    w1t = W1.transpose(2, 3, 1, 0).reshape(50, 30)
    w2t = W2.transpose(2, 3, 1, 0).reshape(270, 100)
    w3t = W3.transpose(2, 3, 1, 0).reshape(900, 200)
    out = pl.pallas_call(
        _net_body,
        out_shape=jax.ShapeDtypeStruct((32, 32, 200), jnp.float32),
        scratch_shapes=[
            pltpu.VMEM((66, 66, 30), jnp.float32),
            pltpu.VMEM((34, 34, 100), jnp.float32),
        ],
    )(cols1, w1t, w2t, w3t)
    return jnp.moveaxis(out, -1, 0)
